# baseline (speedup 1.0000x reference)
import functools

import jax
import jax.numpy as jnp
from jax import lax
from jax.experimental import pallas as pl
from jax.experimental.pallas import tpu as pltpu
from jax.experimental.pallas import tpu_sc as plsc

N = 10000
D = 256
H = 256
C = 2
G = 64
E = 160000

NC = 2
NS = 16
HW = H // NC
K = 128
NBUF = 2
NPAIR = 40
NSEG = 4
SEGC = 20
EPT = 10240
EPAD = EPT * NS
NCHUNK = EPT // K
RPT = 632
NPAD = RPT * NS
DW = 16


_TC_PARAMS = pltpu.CompilerParams(vmem_limit_bytes=100 * 1024 * 1024)


def _sc_mesh():
    return plsc.VectorSubcoreMesh(core_axis_name="c", subcore_axis_name="s")


def _sc_agg(hlflat, srcs2, dst3):
    zin = jnp.zeros((NPAD, HW), jnp.float32)

    @functools.partial(
        pl.kernel,
        out_type=jax.ShapeDtypeStruct((NC * NPAD, HW), jnp.float32),
        mesh=_sc_mesh(),
        scratch_types=[
            pltpu.VMEM_SHARED((NPAD, HW), jnp.float32),
            pltpu.VMEM((NCHUNK, K), jnp.int32),
            [pltpu.VMEM((SEGC * K,), jnp.int32) for _ in range(2)],
            [pltpu.VMEM((K, HW), jnp.float32) for _ in range(NBUF)],
            [pltpu.SemaphoreType.DMA for _ in range(NBUF)],
        ],
    )
    def k(hl_hbm, srcs_hbm, dst_hbm, zin_hbm, agg_hbm,
          agg_sh, didx, sidxs, rowss, gsems):
        c = lax.axis_index("c")
        s = lax.axis_index("s")
        r0 = s * RPT
        w = c * NS + s
        srow0 = w * NSEG
        pltpu.sync_copy(zin_hbm.at[pl.ds(r0, RPT)], agg_sh.at[pl.ds(r0, RPT)])
        pltpu.sync_copy(dst_hbm.at[s], didx)
        plsc.subcore_barrier()

        pltpu.sync_copy(srcs_hbm.at[srow0], sidxs[0])
        pltpu.sync_copy(srcs_hbm.at[srow0 + 1], sidxs[1])
        for b in range(NBUF):
            pltpu.async_copy(
                hl_hbm.at[sidxs[0].at[pl.ds(b * K, K)]], rowss[b], gsems[b])

        def chunk_pair(j0, sref_cur, i0, sref_next):
            for b in range(NBUF):
                pltpu.make_async_copy(
                    hl_hbm.at[pl.ds(0, K)], rowss[b], gsems[b]).wait()
                pltpu.sync_copy(rowss[b], agg_sh.at[didx.at[j0 + b]], add=True)
                if sref_next is not None:
                    pltpu.async_copy(
                        hl_hbm.at[sref_next.at[pl.ds(i0 + b * K, K)]],
                        rowss[b], gsems[b])

        for seg in range(NSEG):
            sb = seg % 2
            j00 = seg * SEGC

            def pair_body(p, carry):
                j0 = j00 + p * 2
                chunk_pair(j0, sidxs[sb], (p * 2 + 2) * K, sidxs[sb])
                return carry

            lax.fori_loop(0, SEGC // 2 - 1, pair_body, 0)
            if seg < NSEG - 1:
                chunk_pair(j00 + SEGC - 2, sidxs[sb], 0, sidxs[1 - sb])
                if seg < NSEG - 2:
                    pltpu.sync_copy(srcs_hbm.at[srow0 + seg + 2], sidxs[sb])
            else:
                chunk_pair(j00 + SEGC - 2, sidxs[sb], 0, None)

        plsc.subcore_barrier()
        pltpu.sync_copy(agg_sh.at[pl.ds(r0, RPT)],
                        agg_hbm.at[pl.ds(c * NPAD + r0, RPT)])

    return k(hlflat, srcs2, dst3, zin)


def _sc_deg(dstd3):
    zin = jnp.zeros((NPAD, HW), jnp.float32)
    ones = jnp.ones((K, HW), jnp.float32)
    nch2 = NCHUNK // 2

    @functools.partial(
        pl.kernel,
        out_type=jax.ShapeDtypeStruct((NC * NPAD, HW), jnp.float32),
        mesh=_sc_mesh(),
        scratch_types=[
            pltpu.VMEM_SHARED((NPAD, HW), jnp.float32),
            pltpu.VMEM((nch2, K), jnp.int32),
            pltpu.VMEM((K, HW), jnp.float32),
        ],
    )
    def k(dst_hbm, zin_hbm, ones_hbm, deg_hbm, deg_sh, didx, ones_v):
        c = lax.axis_index("c")
        s = lax.axis_index("s")
        r0 = s * RPT
        w = c * NS + s
        pltpu.sync_copy(zin_hbm.at[pl.ds(r0, RPT)], deg_sh.at[pl.ds(r0, RPT)])
        pltpu.sync_copy(ones_hbm, ones_v)
        pltpu.sync_copy(dst_hbm.at[w], didx)
        plsc.subcore_barrier()

        def chunk(j, carry):
            pltpu.sync_copy(ones_v, deg_sh.at[didx.at[j]], add=True)
            return carry

        lax.fori_loop(0, nch2, chunk, 0)
        plsc.subcore_barrier()
        pltpu.sync_copy(deg_sh.at[pl.ds(r0, RPT)],
                        deg_hbm.at[pl.ds(c * NPAD + r0, RPT)])

    return k(dstd3, zin, ones)


def _tc_front(x, Wl, Wr):
    def body(x_ref, wl_ref, wr_ref, hl_ref, hr_ref):
        xv = x_ref[...].astype(jnp.bfloat16)
        wl = wl_ref[...].astype(jnp.bfloat16)
        wr = wr_ref[...].astype(jnp.bfloat16)
        hl = jnp.dot(xv, wl, preferred_element_type=jnp.float32)
        hr_ref[...] = jnp.dot(xv, wr, preferred_element_type=jnp.float32)
        hl_ref[0:N, :] = hl[:, 0:HW]
        hl_ref[N:2 * N, :] = hl[:, HW:H]

    return pl.pallas_call(
        body,
        out_shape=[
            jax.ShapeDtypeStruct((NC * N, HW), jnp.float32),
            jax.ShapeDtypeStruct((N, H), jnp.float32),
        ],
        compiler_params=_TC_PARAMS,
    )(x, Wl, Wr)


def _combine(agg_ref, deg_ref, hr_ref, bl_ref, g_ref, be_ref):
    a0 = agg_ref[0:N, :]
    a1 = agg_ref[NPAD:NPAD + N, :]
    aggc = jnp.concatenate([a0, a1], axis=1)
    degv = deg_ref[0:N, 0:1] + deg_ref[NPAD:NPAD + N, 0:1]
    degv = jnp.maximum(degv, 1.0)
    t = aggc / degv + bl_ref[...][None, :] + hr_ref[...]
    m = jnp.mean(t, axis=0, keepdims=True)
    v = jnp.mean(t * t, axis=0, keepdims=True) - m * m
    h = (t - m) * lax.rsqrt(v + 1e-5) * g_ref[...][None, :] + be_ref[...][None, :]
    return jnp.maximum(h, 0.0)


def _tc_mid(agg, deg, hr, bl, g, be, Wl, Wr):
    def body(agg_ref, deg_ref, hr_ref, bl_ref, g_ref, be_ref, wl_ref, wr_ref,
             hl_ref, hr2_ref):
        h = _combine(agg_ref, deg_ref, hr_ref, bl_ref, g_ref, be_ref)
        hb = h.astype(jnp.bfloat16)
        wl = wl_ref[...].astype(jnp.bfloat16)
        wr = wr_ref[...].astype(jnp.bfloat16)
        hl = jnp.dot(hb, wl, preferred_element_type=jnp.float32)
        hr2_ref[...] = jnp.dot(hb, wr, preferred_element_type=jnp.float32)
        hl_ref[0:N, :] = hl[:, 0:HW]
        hl_ref[N:2 * N, :] = hl[:, HW:H]

    return pl.pallas_call(
        body,
        out_shape=[
            jax.ShapeDtypeStruct((NC * N, HW), jnp.float32),
            jax.ShapeDtypeStruct((N, H), jnp.float32),
        ],
        compiler_params=_TC_PARAMS,
    )(agg, deg, hr, bl, g, be, Wl, Wr)


def _tc_final(agg, deg, hr, bl, g, be, batch, linWp, linbp):
    def body(agg_ref, deg_ref, hr_ref, bl_ref, g_ref, be_ref, b_ref, w_ref,
             wb_ref, out_ref):
        h = _combine(agg_ref, deg_ref, hr_ref, bl_ref, g_ref, be_ref)
        bb = b_ref[...]
        gids = lax.broadcasted_iota(jnp.int32, (G, N), 0)
        oh = (bb[None, :] == gids).astype(jnp.float32)
        psum = jnp.dot(oh.astype(jnp.bfloat16), h.astype(jnp.bfloat16),
                       preferred_element_type=jnp.float32)
        cnt = jnp.sum(oh, axis=1, keepdims=True)
        pooled = psum / jnp.maximum(cnt, 1.0)
        out_ref[...] = (
            jnp.dot(pooled, w_ref[...], preferred_element_type=jnp.float32)
            + wb_ref[...][None, :]
        )

    return pl.pallas_call(
        body,
        out_shape=jax.ShapeDtypeStruct((G, 128), jnp.float32),
        compiler_params=_TC_PARAMS,
    )(agg, deg, hr, bl, g, be, batch, linWp, linbp)


def kernel(x, edge_index, batch, Wl1, bl1, Wr1, g1, be1, Wl2, bl2, Wr2, g2,
           be2, Wl3, bl3, Wr3, g3, be3, linW, linb):
    src = edge_index[0]
    dst = edge_index[1]
    src_p = jnp.concatenate([src, jnp.zeros((EPAD - E,), jnp.int32)])
    dst_p = jnp.concatenate([dst, jnp.full((EPAD - E,), N, jnp.int32)])
    srcs2 = jnp.concatenate([src_p, src_p + N]).reshape(NC * NS * NSEG, SEGC * K)
    dst3 = dst_p.reshape(NS, NCHUNK, K)
    dstd3 = dst_p.reshape(NC * NS, NCHUNK // 2, K)

    linWp = jnp.zeros((H, 128), jnp.float32).at[:, :C].set(linW)
    linbp = jnp.zeros((128,), jnp.float32).at[:C].set(linb)

    deg = _sc_deg(dstd3)
    hl1, hr1 = _tc_front(x, Wl1, Wr1)
    agg1 = _sc_agg(hl1, srcs2, dst3)
    hl2, hr2 = _tc_mid(agg1, deg, hr1, bl1, g1, be1, Wl2, Wr2)
    agg2 = _sc_agg(hl2, srcs2, dst3)
    hl3, hr3 = _tc_mid(agg2, deg, hr2, bl2, g2, be2, Wl3, Wr3)
    agg3 = _sc_agg(hl3, srcs2, dst3)
    outp = _tc_final(agg3, deg, hr3, bl3, g3, be3, batch, linWp, linbp)
    return outp[:, :C]


# speedup vs baseline: 1.0088x; 1.0088x over previous
import functools

import jax
import jax.numpy as jnp
from jax import lax
from jax.experimental import pallas as pl
from jax.experimental.pallas import tpu as pltpu
from jax.experimental.pallas import tpu_sc as plsc

N = 10000
D = 256
H = 256
C = 2
G = 64
E = 160000

NC = 2
NS = 16
HW = H // NC
K = 128
NBUF = 2
NPAIR = 40
EPT = 10240
EPAD = EPT * NS
NCHUNK = EPT // K
RPT = 632
NPAD = RPT * NS
DW = 16


_TC_PARAMS = pltpu.CompilerParams(vmem_limit_bytes=100 * 1024 * 1024)


def _sc_mesh():
    return plsc.VectorSubcoreMesh(core_axis_name="c", subcore_axis_name="s")


def _sc_agg(hlflat, srcs2, dst3):
    zin = jnp.zeros((NPAD, HW), jnp.float32)

    @functools.partial(
        pl.kernel,
        out_type=jax.ShapeDtypeStruct((NC * NPAD, HW), jnp.float32),
        mesh=_sc_mesh(),
        scratch_types=[
            pltpu.VMEM_SHARED((NPAD, HW), jnp.float32),
            pltpu.VMEM((NCHUNK, K), jnp.int32),
            [pltpu.VMEM((2 * K,), jnp.int32) for _ in range(2)],
            [pltpu.VMEM((K, HW), jnp.float32) for _ in range(NBUF)],
            [pltpu.SemaphoreType.DMA for _ in range(NBUF)],
            [pltpu.SemaphoreType.DMA for _ in range(NBUF)],
        ],
    )
    def k(hl_hbm, srcs_hbm, dst_hbm, zin_hbm, agg_hbm,
          agg_sh, didx, sidxp, rowss, gsems, ssems):
        c = lax.axis_index("c")
        s = lax.axis_index("s")
        r0 = s * RPT
        w = c * NS + s
        prow0 = w * NPAIR
        pltpu.sync_copy(zin_hbm.at[pl.ds(r0, RPT)], agg_sh.at[pl.ds(r0, RPT)])
        pltpu.sync_copy(dst_hbm.at[s], didx)
        plsc.subcore_barrier()

        pltpu.sync_copy(srcs_hbm.at[prow0], sidxp[0])
        pltpu.sync_copy(srcs_hbm.at[prow0 + 1], sidxp[1])
        for b in range(NBUF):
            pltpu.async_copy(
                hl_hbm.at[sidxp[0].at[pl.ds(b * K, K)]], rowss[b], gsems[b])

        def halfstep(j0, half, prefetch, reload):
            for b in range(NBUF):
                pltpu.make_async_copy(
                    hl_hbm.at[pl.ds(0, K)], rowss[b], gsems[b]).wait()
                pltpu.sync_copy(rowss[b], agg_sh.at[didx.at[j0 + b]], add=True)
                if prefetch:
                    pltpu.async_copy(
                        hl_hbm.at[sidxp[1 - half].at[pl.ds(b * K, K)]],
                        rowss[b], gsems[b])
            if reload:
                pltpu.sync_copy(srcs_hbm.at[prow0 + (j0 // 2) + 2], sidxp[half])

        def quad(q, carry):
            j0 = q * 4
            halfstep(j0, 0, True, True)
            halfstep(j0 + 2, 1, True, True)
            return carry

        lax.fori_loop(0, NCHUNK // 4 - 1, quad, 0)
        halfstep(NCHUNK - 4, 0, True, False)
        halfstep(NCHUNK - 2, 1, False, False)

        plsc.subcore_barrier()
        pltpu.sync_copy(agg_sh.at[pl.ds(r0, RPT)],
                        agg_hbm.at[pl.ds(c * NPAD + r0, RPT)])

    return k(hlflat, srcs2, dst3, zin)


def _sc_deg(dstd3):
    zin = jnp.zeros((NPAD, HW), jnp.float32)
    ones = jnp.ones((K, HW), jnp.float32)
    nch2 = NCHUNK // 2

    @functools.partial(
        pl.kernel,
        out_type=jax.ShapeDtypeStruct((NC * NPAD, HW), jnp.float32),
        mesh=_sc_mesh(),
        scratch_types=[
            pltpu.VMEM_SHARED((NPAD, HW), jnp.float32),
            pltpu.VMEM((nch2, K), jnp.int32),
            pltpu.VMEM((K, HW), jnp.float32),
        ],
    )
    def k(dst_hbm, zin_hbm, ones_hbm, deg_hbm, deg_sh, didx, ones_v):
        c = lax.axis_index("c")
        s = lax.axis_index("s")
        r0 = s * RPT
        w = c * NS + s
        pltpu.sync_copy(zin_hbm.at[pl.ds(r0, RPT)], deg_sh.at[pl.ds(r0, RPT)])
        pltpu.sync_copy(ones_hbm, ones_v)
        pltpu.sync_copy(dst_hbm.at[w], didx)
        plsc.subcore_barrier()

        def chunk(j, carry):
            pltpu.sync_copy(ones_v, deg_sh.at[didx.at[j]], add=True)
            return carry

        lax.fori_loop(0, nch2, chunk, 0)
        plsc.subcore_barrier()
        pltpu.sync_copy(deg_sh.at[pl.ds(r0, RPT)],
                        deg_hbm.at[pl.ds(c * NPAD + r0, RPT)])

    return k(dstd3, zin, ones)


def _tc_front(x, Wl, Wr):
    def body(x_ref, wl_ref, wr_ref, hl_ref, hr_ref):
        xv = x_ref[...].astype(jnp.bfloat16)
        wl = wl_ref[...].astype(jnp.bfloat16)
        wr = wr_ref[...].astype(jnp.bfloat16)
        hl = jnp.dot(xv, wl, preferred_element_type=jnp.float32)
        hr_ref[...] = jnp.dot(xv, wr, preferred_element_type=jnp.float32)
        hl_ref[0:N, :] = hl[:, 0:HW]
        hl_ref[N:2 * N, :] = hl[:, HW:H]

    return pl.pallas_call(
        body,
        out_shape=[
            jax.ShapeDtypeStruct((NC * N, HW), jnp.float32),
            jax.ShapeDtypeStruct((N, H), jnp.float32),
        ],
        compiler_params=_TC_PARAMS,
    )(x, Wl, Wr)


def _combine(agg_ref, deg_ref, hr_ref, bl_ref, g_ref, be_ref):
    a0 = agg_ref[0:N, :]
    a1 = agg_ref[NPAD:NPAD + N, :]
    aggc = jnp.concatenate([a0, a1], axis=1)
    degv = deg_ref[0:N, 0:1] + deg_ref[NPAD:NPAD + N, 0:1]
    degv = jnp.maximum(degv, 1.0)
    t = aggc / degv + bl_ref[...][None, :] + hr_ref[...]
    m = jnp.mean(t, axis=0, keepdims=True)
    v = jnp.mean(t * t, axis=0, keepdims=True) - m * m
    h = (t - m) * lax.rsqrt(v + 1e-5) * g_ref[...][None, :] + be_ref[...][None, :]
    return jnp.maximum(h, 0.0)


def _tc_mid(agg, deg, hr, bl, g, be, Wl, Wr):
    def body(agg_ref, deg_ref, hr_ref, bl_ref, g_ref, be_ref, wl_ref, wr_ref,
             hl_ref, hr2_ref):
        h = _combine(agg_ref, deg_ref, hr_ref, bl_ref, g_ref, be_ref)
        hb = h.astype(jnp.bfloat16)
        wl = wl_ref[...].astype(jnp.bfloat16)
        wr = wr_ref[...].astype(jnp.bfloat16)
        hl = jnp.dot(hb, wl, preferred_element_type=jnp.float32)
        hr2_ref[...] = jnp.dot(hb, wr, preferred_element_type=jnp.float32)
        hl_ref[0:N, :] = hl[:, 0:HW]
        hl_ref[N:2 * N, :] = hl[:, HW:H]

    return pl.pallas_call(
        body,
        out_shape=[
            jax.ShapeDtypeStruct((NC * N, HW), jnp.float32),
            jax.ShapeDtypeStruct((N, H), jnp.float32),
        ],
        compiler_params=_TC_PARAMS,
    )(agg, deg, hr, bl, g, be, Wl, Wr)


def _tc_final(agg, deg, hr, bl, g, be, batch, linWp, linbp):
    def body(agg_ref, deg_ref, hr_ref, bl_ref, g_ref, be_ref, b_ref, w_ref,
             wb_ref, out_ref):
        h = _combine(agg_ref, deg_ref, hr_ref, bl_ref, g_ref, be_ref)
        bb = b_ref[...]
        gids = lax.broadcasted_iota(jnp.int32, (G, N), 0)
        oh = (bb[None, :] == gids).astype(jnp.float32)
        psum = jnp.dot(oh.astype(jnp.bfloat16), h.astype(jnp.bfloat16),
                       preferred_element_type=jnp.float32)
        cnt = jnp.sum(oh, axis=1, keepdims=True)
        pooled = psum / jnp.maximum(cnt, 1.0)
        out_ref[...] = (
            jnp.dot(pooled, w_ref[...], preferred_element_type=jnp.float32)
            + wb_ref[...][None, :]
        )

    return pl.pallas_call(
        body,
        out_shape=jax.ShapeDtypeStruct((G, 128), jnp.float32),
        compiler_params=_TC_PARAMS,
    )(agg, deg, hr, bl, g, be, batch, linWp, linbp)


def kernel(x, edge_index, batch, Wl1, bl1, Wr1, g1, be1, Wl2, bl2, Wr2, g2,
           be2, Wl3, bl3, Wr3, g3, be3, linW, linb):
    src = edge_index[0]
    dst = edge_index[1]
    src_p = jnp.concatenate([src, jnp.zeros((EPAD - E,), jnp.int32)])
    dst_p = jnp.concatenate([dst, jnp.full((EPAD - E,), N, jnp.int32)])
    srcs2 = jnp.concatenate([src_p, src_p + N]).reshape(NC * NS * NPAIR, 2 * K)
    dst3 = dst_p.reshape(NS, NCHUNK, K)
    dstd3 = dst_p.reshape(NC * NS, NCHUNK // 2, K)

    linWp = jnp.zeros((H, 128), jnp.float32).at[:, :C].set(linW)
    linbp = jnp.zeros((128,), jnp.float32).at[:C].set(linb)

    deg = _sc_deg(dstd3)
    hl1, hr1 = _tc_front(x, Wl1, Wr1)
    agg1 = _sc_agg(hl1, srcs2, dst3)
    hl2, hr2 = _tc_mid(agg1, deg, hr1, bl1, g1, be1, Wl2, Wr2)
    agg2 = _sc_agg(hl2, srcs2, dst3)
    hl3, hr3 = _tc_mid(agg2, deg, hr2, bl2, g2, be2, Wl3, Wr3)
    agg3 = _sc_agg(hl3, srcs2, dst3)
    outp = _tc_final(agg3, deg, hr3, bl3, g3, be3, batch, linWp, linbp)
    return outp[:, :C]
